# baseline (speedup 1.0000x reference)
import functools

import jax
import jax.numpy as jnp
from jax import lax
from jax.experimental import pallas as pl
from jax.experimental.pallas import tpu as pltpu
from jax.experimental.pallas import tpu_sc as plsc

B = 4
SEQ = 4096
HID = 2048
LANES = 16
NCHUNK = SEQ // LANES


def _sc_body(pos_hbm, idx_hbm, pos_v, idx16_v):
    nc = 2
    wid = lax.axis_index("s") * nc + lax.axis_index("c")

    @pl.when(wid < B)
    def _():
        pltpu.sync_copy(pos_hbm.at[wid], pos_v)
        lane = lax.iota(jnp.int32, LANES)

        def body(i, acc):
            v = pos_v[pl.ds(i * LANES, LANES)]
            key = v * SEQ + (i * LANES + lane)
            return jnp.minimum(acc, key)

        acc = lax.fori_loop(
            0, NCHUNK, body,
            jnp.full((LANES,), jnp.int32(2**30), dtype=jnp.int32),
            unroll=8,
        )
        m = jnp.min(acc)
        idx16_v[...] = jnp.full((LANES,), m & (SEQ - 1), dtype=jnp.int32)
        pltpu.sync_copy(idx16_v, idx_hbm.at[wid])


_sc_argmin = functools.partial(
    pl.kernel,
    out_type=[jax.ShapeDtypeStruct((B, LANES), jnp.int32)],
    mesh=plsc.VectorSubcoreMesh(core_axis_name="c", subcore_axis_name="s"),
    compiler_params=pltpu.CompilerParams(needs_layout_passes=False),
    scratch_types=[
        pltpu.VMEM((SEQ,), jnp.int32),
        pltpu.VMEM((LANES,), jnp.int32),
    ],
)(_sc_body)


BN = 256


NCHUNKS = 8
CH = HID // NCHUNKS


def _tc_body(pos_ref, hs_ref, w_ref, o_ref, oidx_ref, w_v, tok_v, wsem, gsem):
    for c in range(NCHUNKS):
        pltpu.make_async_copy(
            w_ref.at[pl.ds(c * CH, CH)], w_v.at[pl.ds(c * CH, CH)], wsem.at[c]
        ).start()

    pos = pos_ref[...]
    col = lax.broadcasted_iota(jnp.int32, (B, SEQ), 1)
    key = pos * SEQ + col
    for b in range(B):
        idx = jnp.min(key[b]) & (SEQ - 1)
        oidx_ref[0, b] = idx
        pltpu.make_async_copy(hs_ref.at[b, idx], tok_v.at[b], gsem).start()
    for b in range(B):
        pltpu.make_async_copy(hs_ref.at[b, 0], tok_v.at[b], gsem).wait()

    tok = tok_v[...]
    for c in range(NCHUNKS):
        pltpu.make_async_copy(
            w_ref.at[pl.ds(c * CH, CH)], w_v.at[pl.ds(c * CH, CH)], wsem.at[c]
        ).wait()
        o_ref[:, pl.ds(c * CH, CH)] = lax.dot_general(
            tok, w_v[pl.ds(c * CH, CH), :],
            (((1,), (1,)), ((), ())),
            preferred_element_type=jnp.float32,
        )


def _tc_enhanced(pos, hs, W):
    return pl.pallas_call(
        _tc_body,
        in_specs=[
            pl.BlockSpec((B, SEQ), lambda: (0, 0)),
            pl.BlockSpec(memory_space=pl.ANY),
            pl.BlockSpec(memory_space=pl.ANY),
        ],
        out_specs=[
            pl.BlockSpec((B, HID), lambda: (0, 0)),
            pl.BlockSpec(memory_space=pltpu.SMEM),
        ],
        out_shape=[
            jax.ShapeDtypeStruct((B, HID), jnp.float32),
            jax.ShapeDtypeStruct((1, B), jnp.int32),
        ],
        scratch_shapes=[
            pltpu.VMEM((HID, HID), jnp.float32),
            pltpu.VMEM((B, HID), jnp.float32),
            pltpu.SemaphoreType.DMA((NCHUNKS,)),
            pltpu.SemaphoreType.DMA,
        ],
    )(pos, hs, W)


def kernel(hidden_states, positions, W):
    pos = positions.astype(jnp.int32)
    enhanced, idx = _tc_enhanced(pos, hidden_states, W)
    return (enhanced, idx[0])


# speedup vs baseline: 1.2562x; 1.2562x over previous
import functools

import jax
import jax.numpy as jnp
from jax import lax
from jax.experimental import pallas as pl
from jax.experimental.pallas import tpu as pltpu
from jax.experimental.pallas import tpu_sc as plsc

B = 4
SEQ = 4096
HID = 2048
LANES = 16
NCHUNK = SEQ // LANES


def _sc_body(pos_hbm, idx_hbm, pos_v, idx16_v):
    nc = 2
    wid = lax.axis_index("s") * nc + lax.axis_index("c")

    @pl.when(wid < B)
    def _():
        pltpu.sync_copy(pos_hbm.at[wid], pos_v)
        lane = lax.iota(jnp.int32, LANES)

        def body(i, acc):
            v = pos_v[pl.ds(i * LANES, LANES)]
            key = v * SEQ + (i * LANES + lane)
            return jnp.minimum(acc, key)

        acc = lax.fori_loop(
            0, NCHUNK, body,
            jnp.full((LANES,), jnp.int32(2**30), dtype=jnp.int32),
            unroll=8,
        )
        m = jnp.min(acc)
        idx16_v[...] = jnp.full((LANES,), m & (SEQ - 1), dtype=jnp.int32)
        pltpu.sync_copy(idx16_v, idx_hbm.at[wid])


_sc_argmin = functools.partial(
    pl.kernel,
    out_type=[jax.ShapeDtypeStruct((B, LANES), jnp.int32)],
    mesh=plsc.VectorSubcoreMesh(core_axis_name="c", subcore_axis_name="s"),
    compiler_params=pltpu.CompilerParams(needs_layout_passes=False),
    scratch_types=[
        pltpu.VMEM((SEQ,), jnp.int32),
        pltpu.VMEM((LANES,), jnp.int32),
    ],
)(_sc_body)


BN = 256


NCHUNKS = 4
CH = HID // NCHUNKS


def _tc_body(pos_ref, hs_ref, w_ref, o_ref, oidx_ref, w_v, tok_v, wsem, gsem):
    for c in range(NCHUNKS):
        pltpu.make_async_copy(
            w_ref.at[pl.ds(c * CH, CH)], w_v.at[pl.ds(c * CH, CH)], wsem.at[c]
        ).start()

    pos = pos_ref[...]
    col = lax.broadcasted_iota(jnp.int32, (B, SEQ), 1)
    key = pos * SEQ + col
    for b in range(B):
        idx = jnp.min(key[b]) & (SEQ - 1)
        oidx_ref[0, b] = idx
        pltpu.make_async_copy(hs_ref.at[b, idx], tok_v.at[b], gsem).start()
    for b in range(B):
        pltpu.make_async_copy(hs_ref.at[b, 0], tok_v.at[b], gsem).wait()

    tok = tok_v[...]
    for c in range(NCHUNKS):
        pltpu.make_async_copy(
            w_ref.at[pl.ds(c * CH, CH)], w_v.at[pl.ds(c * CH, CH)], wsem.at[c]
        ).wait()
        o_ref[:, pl.ds(c * CH, CH)] = lax.dot_general(
            tok, w_v[pl.ds(c * CH, CH), :],
            (((1,), (1,)), ((), ())),
            preferred_element_type=jnp.float32,
        )


def _tc_enhanced(pos, hs, W):
    return pl.pallas_call(
        _tc_body,
        in_specs=[
            pl.BlockSpec((B, SEQ), lambda: (0, 0)),
            pl.BlockSpec(memory_space=pl.ANY),
            pl.BlockSpec(memory_space=pl.ANY),
        ],
        out_specs=[
            pl.BlockSpec((B, HID), lambda: (0, 0)),
            pl.BlockSpec(memory_space=pltpu.SMEM),
        ],
        out_shape=[
            jax.ShapeDtypeStruct((B, HID), jnp.float32),
            jax.ShapeDtypeStruct((1, B), jnp.int32),
        ],
        scratch_shapes=[
            pltpu.VMEM((HID, HID), jnp.float32),
            pltpu.VMEM((B, HID), jnp.float32),
            pltpu.SemaphoreType.DMA((NCHUNKS,)),
            pltpu.SemaphoreType.DMA,
        ],
    )(pos, hs, W)


def kernel(hidden_states, positions, W):
    pos = positions.astype(jnp.int32)
    enhanced, idx = _tc_enhanced(pos, hidden_states, W)
    return (enhanced, idx[0])


# speedup vs baseline: 1.2877x; 1.0251x over previous
import functools

import jax
import jax.numpy as jnp
from jax import lax
from jax.experimental import pallas as pl
from jax.experimental.pallas import tpu as pltpu
from jax.experimental.pallas import tpu_sc as plsc

B = 4
SEQ = 4096
HID = 2048
LANES = 16
NCHUNK = SEQ // LANES


def _sc_body(pos_hbm, idx_hbm, pos_v, idx16_v):
    nc = 2
    wid = lax.axis_index("s") * nc + lax.axis_index("c")

    @pl.when(wid < B)
    def _():
        pltpu.sync_copy(pos_hbm.at[wid], pos_v)
        lane = lax.iota(jnp.int32, LANES)

        def body(i, acc):
            v = pos_v[pl.ds(i * LANES, LANES)]
            key = v * SEQ + (i * LANES + lane)
            return jnp.minimum(acc, key)

        acc = lax.fori_loop(
            0, NCHUNK, body,
            jnp.full((LANES,), jnp.int32(2**30), dtype=jnp.int32),
            unroll=8,
        )
        m = jnp.min(acc)
        idx16_v[...] = jnp.full((LANES,), m & (SEQ - 1), dtype=jnp.int32)
        pltpu.sync_copy(idx16_v, idx_hbm.at[wid])


_sc_argmin = functools.partial(
    pl.kernel,
    out_type=[jax.ShapeDtypeStruct((B, LANES), jnp.int32)],
    mesh=plsc.VectorSubcoreMesh(core_axis_name="c", subcore_axis_name="s"),
    compiler_params=pltpu.CompilerParams(needs_layout_passes=False),
    scratch_types=[
        pltpu.VMEM((SEQ,), jnp.int32),
        pltpu.VMEM((LANES,), jnp.int32),
    ],
)(_sc_body)


BN = 256


NCHUNKS = 2
CH = HID // NCHUNKS


def _tc_body(pos_ref, hs_ref, w_ref, o_ref, oidx_ref, w_v, tok_v, wsem, gsem):
    for c in range(NCHUNKS):
        pltpu.make_async_copy(
            w_ref.at[pl.ds(c * CH, CH)], w_v.at[pl.ds(c * CH, CH)], wsem.at[c]
        ).start()

    pos = pos_ref[...]
    col = lax.broadcasted_iota(jnp.int32, (B, SEQ), 1)
    key = pos * SEQ + col
    for b in range(B):
        idx = jnp.min(key[b]) & (SEQ - 1)
        oidx_ref[0, b] = idx
        pltpu.make_async_copy(hs_ref.at[b, idx], tok_v.at[b], gsem).start()
    for b in range(B):
        pltpu.make_async_copy(hs_ref.at[b, 0], tok_v.at[b], gsem).wait()

    tok = tok_v[...]
    for c in range(NCHUNKS):
        pltpu.make_async_copy(
            w_ref.at[pl.ds(c * CH, CH)], w_v.at[pl.ds(c * CH, CH)], wsem.at[c]
        ).wait()
        o_ref[:, pl.ds(c * CH, CH)] = lax.dot_general(
            tok, w_v[pl.ds(c * CH, CH), :],
            (((1,), (1,)), ((), ())),
            preferred_element_type=jnp.float32,
        )


def _tc_enhanced(pos, hs, W):
    return pl.pallas_call(
        _tc_body,
        in_specs=[
            pl.BlockSpec((B, SEQ), lambda: (0, 0)),
            pl.BlockSpec(memory_space=pl.ANY),
            pl.BlockSpec(memory_space=pl.ANY),
        ],
        out_specs=[
            pl.BlockSpec((B, HID), lambda: (0, 0)),
            pl.BlockSpec(memory_space=pltpu.SMEM),
        ],
        out_shape=[
            jax.ShapeDtypeStruct((B, HID), jnp.float32),
            jax.ShapeDtypeStruct((1, B), jnp.int32),
        ],
        scratch_shapes=[
            pltpu.VMEM((HID, HID), jnp.float32),
            pltpu.VMEM((B, HID), jnp.float32),
            pltpu.SemaphoreType.DMA((NCHUNKS,)),
            pltpu.SemaphoreType.DMA,
        ],
    )(pos, hs, W)


def kernel(hidden_states, positions, W):
    pos = positions.astype(jnp.int32)
    enhanced, idx = _tc_enhanced(pos, hidden_states, W)
    return (enhanced, idx[0])
